# Initial kernel scaffold; baseline (speedup 1.0000x reference)
import jax
import jax.numpy as jnp
from jax.experimental import pallas as pl


def kernel(x, edge_index, batch, W1, att_src1, att_dst1, b1, W2, att_src2, att_dst2, b2):
    raise NotImplementedError("write your pallas kernel here")


# speedup vs baseline: 28.6542x; 28.6542x over previous
import dataclasses
import functools

import jax
import jax.numpy as jnp
from jax import lax
from jax.experimental import pallas as pl
from jax.experimental.pallas import tpu as pltpu
from jax.experimental.pallas import tpu_sc as plsc

_NC = 2
_NS = 16
_NW = _NC * _NS
_B = 80
_BIG_NEG = -1e30
_F32 = jnp.float32
_HIGH = lax.Precision.HIGHEST


def _tc_prep1(x, w1, a1s, a1d):
  n = x.shape[0]
  f = w1.shape[1]

  def body(x_ref, w_ref, as_ref, ad_ref, h_ref, ss_ref, sd_ref):
    h = jnp.dot(x_ref[...], w_ref[...], preferred_element_type=_F32,
                precision=_HIGH)
    h_ref[...] = h
    ss = jnp.dot(h, as_ref[...], preferred_element_type=_F32, precision=_HIGH)
    col = lax.broadcasted_iota(jnp.int32, ss.shape, 1)
    ss_ref[...] = jnp.where(col >= 8, _BIG_NEG, ss)
    sd_ref[...] = jnp.dot(h, ad_ref[...], preferred_element_type=_F32,
                          precision=_HIGH)

  return pl.pallas_call(
      body,
      out_shape=(
          jax.ShapeDtypeStruct((n, f), _F32),
          jax.ShapeDtypeStruct((n, 16), _F32),
          jax.ShapeDtypeStruct((n, 16), _F32),
      ),
  )(x, w1, a1s, a1d)


def _sc_edges(h, ss, sd, src, dst, zeros, heads):
  n, f = h.shape
  ft = f + 16
  e = src.shape[0]
  ep = e // _NW
  nb = ep // _B
  rp = n // _NS
  cph = f // heads
  mesh = plsc.VectorSubcoreMesh(core_axis_name="c", subcore_axis_name="s")
  cp = pltpu.CompilerParams()
  fields = pltpu.CompilerParams.__dataclass_fields__
  if "needs_layout_passes" in fields:
    cp = dataclasses.replace(cp, needs_layout_passes=False)
  if "use_tc_tiling_on_sc" in fields:
    cp = dataclasses.replace(cp, use_tc_tiling_on_sc=False)

  @functools.partial(
      pl.kernel,
      out_type=jax.ShapeDtypeStruct((_NC, n, ft), _F32),
      mesh=mesh,
      compiler_params=cp,
      scratch_types=[
          pltpu.VMEM_SHARED((n, ft), _F32),
          pltpu.VMEM((_B,), jnp.int32),
          pltpu.VMEM((_B,), jnp.int32),
          pltpu.VMEM((_B, f), _F32),
          pltpu.VMEM((_B, 16), _F32),
          pltpu.VMEM((_B, 16), _F32),
          pltpu.VMEM((_B, ft), _F32),
      ],
  )
  def k(h_hbm, ss_hbm, sd_hbm, src_hbm, dst_hbm, z_hbm, out_hbm,
        acc, isrc, idst, hrows, srows, drows, orows):
    cid = lax.axis_index("c")
    sid = lax.axis_index("s")
    wid = cid * _NS + sid

    @pl.when(sid == 0)
    def _zero():
      pltpu.sync_copy(z_hbm, acc)

    plsc.subcore_barrier()

    mask0 = (lax.iota(jnp.int32, 16) < 1).astype(_F32)
    base0 = wid * ep

    @pl.loop(0, nb)
    def _blk(i):
      base = base0 + i * _B
      pltpu.sync_copy(src_hbm.at[pl.ds(base, _B)], isrc)
      pltpu.sync_copy(dst_hbm.at[pl.ds(base, _B)], idst)
      pltpu.sync_copy(h_hbm.at[isrc], hrows)
      pltpu.sync_copy(ss_hbm.at[isrc], srows)
      pltpu.sync_copy(sd_hbm.at[idst], drows)

      @pl.loop(0, _B)
      def _edge(b):
        a = srows[b, :] + drows[b, :]
        w = jnp.exp(jnp.maximum(a, a * 0.2))
        if heads == 1:
          orows[b, pl.ds(f, 16)] = w * mask0
        else:
          orows[b, pl.ds(f, 16)] = w
        for hd in range(heads):
          if heads == 1:
            sv = w
          else:
            idxk = jnp.full((16,), hd, jnp.int32)
            sv = jnp.take_along_axis(w, idxk, axis=0,
                                     mode="promise_in_bounds")
          for c in range(cph // 16):
            off = hd * cph + c * 16
            orows[b, pl.ds(off, 16)] = hrows[b, pl.ds(off, 16)] * sv

      pltpu.sync_copy(orows, acc.at[idst], add=True)

    plsc.subcore_barrier()

    @pl.when(sid == 0)
    def _writeout():
      pltpu.sync_copy(acc, out_hbm.at[cid])

  return k(h, ss, sd, src, dst, zeros)


def _tc_mid(acc1, b1r, w2, a2s, a2d, rep):
  n = acc1.shape[1]
  f = b1r.shape[1]
  f2 = w2.shape[1]
  heads = rep.shape[0]

  rb = 2000

  def body(a_ref, b1_ref, w2_ref, as_ref, ad_ref, rep_ref,
           h2_ref, ss_ref, sd_ref):
    a = a_ref[...]
    s = a[0] + a[1]
    num = s[:, :f]
    den = s[:, f:f + heads]
    r = 1.0 / (den + 1e-16)
    rf = jnp.dot(r, rep_ref[...], preferred_element_type=_F32,
                 precision=_HIGH)
    o1 = num * rf + b1_ref[...]
    h2 = jnp.dot(o1, w2_ref[...], preferred_element_type=_F32,
                 precision=_HIGH)
    h2_ref[...] = h2
    ss_ref[...] = jnp.dot(h2, as_ref[...], preferred_element_type=_F32,
                          precision=_HIGH)
    sd_ref[...] = jnp.dot(h2, ad_ref[...], preferred_element_type=_F32,
                          precision=_HIGH)

  ft = acc1.shape[2]
  return pl.pallas_call(
      body,
      grid=(n // rb,),
      in_specs=[
          pl.BlockSpec((2, rb, ft), lambda i: (0, i, 0)),
          pl.BlockSpec((1, f), lambda i: (0, 0)),
          pl.BlockSpec(w2.shape, lambda i: (0, 0)),
          pl.BlockSpec(a2s.shape, lambda i: (0, 0)),
          pl.BlockSpec(a2d.shape, lambda i: (0, 0)),
          pl.BlockSpec(rep.shape, lambda i: (0, 0)),
      ],
      out_specs=(
          pl.BlockSpec((rb, f2), lambda i: (i, 0)),
          pl.BlockSpec((rb, 16), lambda i: (i, 0)),
          pl.BlockSpec((rb, 16), lambda i: (i, 0)),
      ),
      out_shape=(
          jax.ShapeDtypeStruct((n, f2), _F32),
          jax.ShapeDtypeStruct((n, 16), _F32),
          jax.ShapeDtypeStruct((n, 16), _F32),
      ),
  )(acc1, b1r, w2, a2s, a2d, rep)


def _tc_final(acc2, b2r, batch2d, g):
  n = acc2.shape[1]
  f2 = b2r.shape[1]

  def body(a_ref, b2_ref, bt_ref, o_ref):
    a = a_ref[...]
    s = a[0] + a[1]
    num = s[:, :f2]
    den = s[:, f2:f2 + 1]
    h2o = num * (1.0 / (den + 1e-16)) + b2_ref[...]
    bt = bt_ref[...]
    gi = lax.broadcasted_iota(jnp.int32, (g, n), 0)
    m = (gi == bt).astype(_F32)
    summ = jnp.dot(m, h2o, preferred_element_type=_F32, precision=_HIGH)
    cnt = jnp.sum(m, axis=1, keepdims=True)
    pooled = summ / jnp.maximum(cnt, 1.0)
    mx = jnp.max(pooled, axis=1, keepdims=True)
    ex = jnp.exp(pooled - mx)
    lse = jnp.log(jnp.sum(ex, axis=1, keepdims=True))
    o_ref[...] = pooled - mx - lse

  return pl.pallas_call(
      body,
      out_shape=jax.ShapeDtypeStruct((g, f2), _F32),
  )(acc2, b2r, batch2d)


def _block_diag(att):
  h_, c_ = att.shape[1], att.shape[2]
  eye = jnp.eye(h_, dtype=_F32)
  m = (att[0][:, :, None] * eye[:, None, :]).reshape(h_ * c_, h_)
  return jnp.pad(m, ((0, 0), (0, 16 - h_)))


def kernel(x, edge_index, batch, W1, att_src1, att_dst1, b1,
           W2, att_src2, att_dst2, b2):
  n = x.shape[0]
  f1 = W1.shape[1]
  f2 = W2.shape[1]
  g = 64
  h1n = att_src1.shape[1]

  src = edge_index[0]
  dst = edge_index[1]
  a1s = _block_diag(att_src1)
  a1d = _block_diag(att_dst1)
  a2s = jnp.tile(att_src2[0, 0][:, None], (1, 16)).astype(_F32)
  a2d = jnp.tile(att_dst2[0, 0][:, None], (1, 16)).astype(_F32)
  rep1 = jnp.repeat(jnp.eye(h1n, dtype=_F32), f1 // h1n, axis=1)
  z1 = jnp.zeros((n, f1 + 16), _F32)
  z2 = jnp.zeros((n, f2 + 16), _F32)
  b1r = b1.reshape(1, f1)
  b2r = b2.reshape(1, f2)
  batch2d = batch.reshape(1, n)

  h1, ss1, sd1 = _tc_prep1(x, W1, a1s, a1d)
  acc1 = _sc_edges(h1, ss1, sd1, src, dst, z1, heads=h1n)
  h2, ss2, sd2 = _tc_mid(acc1, b1r, W2, a2s, a2d, rep1)
  acc2 = _sc_edges(h2, ss2, sd2, src, dst, z2, heads=1)
  return _tc_final(acc2, b2r, batch2d, g)


# speedup vs baseline: 55.1717x; 1.9254x over previous
import dataclasses
import functools

import jax
import jax.numpy as jnp
from jax import lax
from jax.experimental import pallas as pl
from jax.experimental.pallas import tpu as pltpu
from jax.experimental.pallas import tpu_sc as plsc

_NC = 2
_NS = 16
_NW = _NC * _NS
_B = 80
_BIG_NEG = -1e30
_F32 = jnp.float32
_HIGH = lax.Precision.HIGHEST


def _tc_prep1(x, w1, a1s, a1d):
  n = x.shape[0]
  f = w1.shape[1]

  def body(x_ref, w_ref, as_ref, ad_ref, h_ref, ss_ref, sd_ref):
    h = jnp.dot(x_ref[...], w_ref[...], preferred_element_type=_F32,
                precision=_HIGH)
    h_ref[...] = h
    ss = jnp.dot(h, as_ref[...], preferred_element_type=_F32, precision=_HIGH)
    col = lax.broadcasted_iota(jnp.int32, ss.shape, 1)
    ss_ref[...] = jnp.where(col >= 8, _BIG_NEG, ss)
    sd_ref[...] = jnp.dot(h, ad_ref[...], preferred_element_type=_F32,
                          precision=_HIGH)

  return pl.pallas_call(
      body,
      out_shape=(
          jax.ShapeDtypeStruct((n, f), _F32),
          jax.ShapeDtypeStruct((n, 16), _F32),
          jax.ShapeDtypeStruct((n, 16), _F32),
      ),
  )(x, w1, a1s, a1d)


def _sc_edges(h, ss, sd, src3, dst3, zeros, heads):
  n, f = h.shape
  ft = f + 16
  nw, nb, bsz = src3.shape
  cph = f // heads
  mesh = plsc.VectorSubcoreMesh(core_axis_name="c", subcore_axis_name="s")
  cp = pltpu.CompilerParams()
  fields = pltpu.CompilerParams.__dataclass_fields__
  if "needs_layout_passes" in fields:
    cp = dataclasses.replace(cp, needs_layout_passes=False)
  if "use_tc_tiling_on_sc" in fields:
    cp = dataclasses.replace(cp, use_tc_tiling_on_sc=False)

  @functools.partial(
      pl.kernel,
      out_type=jax.ShapeDtypeStruct((_NC, n, ft), _F32),
      mesh=mesh,
      compiler_params=cp,
      scratch_types=[
          pltpu.VMEM_SHARED((n, ft), _F32),
          pltpu.VMEM((4, bsz), jnp.int32),
          pltpu.VMEM((4, bsz), jnp.int32),
          pltpu.VMEM((2, bsz, f), _F32),
          pltpu.VMEM((2, bsz, 16), _F32),
          pltpu.VMEM((2, bsz, 16), _F32),
          pltpu.VMEM((2, bsz, ft), _F32),
          pltpu.SemaphoreType.DMA,
          pltpu.SemaphoreType.DMA,
          pltpu.SemaphoreType.DMA,
          pltpu.SemaphoreType.DMA,
          pltpu.SemaphoreType.DMA,
          pltpu.SemaphoreType.DMA,
      ],
  )
  def k(h_hbm, ss_hbm, sd_hbm, src_hbm, dst_hbm, z_hbm, out_hbm,
        acc, isrc, idst, hrows, srows, drows, orows,
        gs0, gs1, sc0, sc1, ix0, ix1):
    cid = lax.axis_index("c")
    sid = lax.axis_index("s")
    wid = cid * _NS + sid
    gsems = (gs0, gs1)
    ssems = (sc0, sc1)
    isems = (ix0, ix1)

    @pl.when(sid == 0)
    def _zero():
      pltpu.sync_copy(z_hbm, acc)

    plsc.subcore_barrier()

    mask0 = (lax.iota(jnp.int32, 16) < 1).astype(_F32)

    def i_descs(b, s):
      j = jnp.bitwise_and(b, 3)
      return (
          pltpu.make_async_copy(src_hbm.at[wid, b], isrc.at[j], isems[s]),
          pltpu.make_async_copy(dst_hbm.at[wid, b], idst.at[j], isems[s]),
      )

    def issue_i(b, s):
      for d in i_descs(b, s):
        d.start()

    def wait_i(b, s):
      for d in i_descs(b, s):
        d.wait()

    def g_descs(b, s):
      j = jnp.bitwise_and(b, 3)
      return (
          pltpu.make_async_copy(h_hbm.at[isrc.at[j]], hrows.at[s], gsems[s]),
          pltpu.make_async_copy(ss_hbm.at[isrc.at[j]], srows.at[s], gsems[s]),
          pltpu.make_async_copy(sd_hbm.at[idst.at[j]], drows.at[s], gsems[s]),
      )

    def issue_g(b, s):
      for d in g_descs(b, s):
        d.start()

    def wait_g(b, s):
      for d in g_descs(b, s):
        d.wait()

    def issue_s(b, s):
      j = jnp.bitwise_and(b, 3)
      pltpu.async_copy(orows.at[s], acc.at[idst.at[j]], ssems[s], add=True)

    def wait_s(b, s):
      j = jnp.bitwise_and(b, 3)
      pltpu.make_async_copy(orows.at[s], acc.at[idst.at[j]], ssems[s]).wait()

    def compute(s):
      @pl.loop(0, bsz, unroll=4)
      def _edge(e):
        a = srows[s, e, :] + drows[s, e, :]
        w = jnp.exp(jnp.maximum(a, a * 0.2))
        if heads == 1:
          orows[s, e, pl.ds(f, 16)] = w * mask0
        else:
          orows[s, e, pl.ds(f, 16)] = w
        for hd in range(heads):
          if heads == 1:
            sv = w
          else:
            idxk = jnp.full((16,), hd, jnp.int32)
            sv = jnp.take_along_axis(w, idxk, axis=0,
                                     mode="promise_in_bounds")
          for c in range(cph // 16):
            off = hd * cph + c * 16
            orows[s, e, pl.ds(off, 16)] = hrows[s, e, pl.ds(off, 16)] * sv

    issue_i(0, 0)
    issue_i(1, 1)
    wait_i(0, 0)
    issue_g(0, 0)
    wait_i(1, 1)
    issue_g(1, 1)

    @pl.loop(0, nb - 1, step=2)
    def _pair(i):
      for p in range(2):
        b = i + p
        wait_g(b, p)

        @pl.when(b >= 2)
        def _():
          wait_s(b - 2, p)

        @pl.when(b + 2 < nb)
        def _():
          issue_i(b + 2, p)

        compute(p)
        issue_s(b, p)

        @pl.when(b + 2 < nb)
        def _():
          wait_i(b + 2, p)
          issue_g(b + 2, p)

    if nb % 2 == 1:
      bt = nb - 1
      wait_g(bt, 0)
      wait_s(bt - 2, 0)
      compute(0)
      issue_s(bt, 0)
      wait_s(bt - 1, 1)
      wait_s(bt, 0)
    else:
      wait_s(nb - 2, 0)
      wait_s(nb - 1, 1)

    plsc.subcore_barrier()

    @pl.when(sid == 0)
    def _writeout():
      pltpu.sync_copy(acc, out_hbm.at[cid])

  return k(h, ss, sd, src3, dst3, zeros)


def _tc_mid(acc1, b1r, w2, a2s, a2d, rep):
  n = acc1.shape[1]
  f = b1r.shape[1]
  f2 = w2.shape[1]
  heads = rep.shape[0]

  rb = 2000

  def body(a_ref, b1_ref, w2_ref, as_ref, ad_ref, rep_ref,
           h2_ref, ss_ref, sd_ref):
    a = a_ref[...]
    s = a[0] + a[1]
    num = s[:, :f]
    den = s[:, f:f + heads]
    r = 1.0 / (den + 1e-16)
    rf = jnp.dot(r, rep_ref[...], preferred_element_type=_F32,
                 precision=_HIGH)
    o1 = num * rf + b1_ref[...]
    h2 = jnp.dot(o1, w2_ref[...], preferred_element_type=_F32,
                 precision=_HIGH)
    h2_ref[...] = h2
    ss_ref[...] = jnp.dot(h2, as_ref[...], preferred_element_type=_F32,
                          precision=_HIGH)
    sd_ref[...] = jnp.dot(h2, ad_ref[...], preferred_element_type=_F32,
                          precision=_HIGH)

  ft = acc1.shape[2]
  return pl.pallas_call(
      body,
      grid=(n // rb,),
      in_specs=[
          pl.BlockSpec((2, rb, ft), lambda i: (0, i, 0)),
          pl.BlockSpec((1, f), lambda i: (0, 0)),
          pl.BlockSpec(w2.shape, lambda i: (0, 0)),
          pl.BlockSpec(a2s.shape, lambda i: (0, 0)),
          pl.BlockSpec(a2d.shape, lambda i: (0, 0)),
          pl.BlockSpec(rep.shape, lambda i: (0, 0)),
      ],
      out_specs=(
          pl.BlockSpec((rb, f2), lambda i: (i, 0)),
          pl.BlockSpec((rb, 16), lambda i: (i, 0)),
          pl.BlockSpec((rb, 16), lambda i: (i, 0)),
      ),
      out_shape=(
          jax.ShapeDtypeStruct((n, f2), _F32),
          jax.ShapeDtypeStruct((n, 16), _F32),
          jax.ShapeDtypeStruct((n, 16), _F32),
      ),
  )(acc1, b1r, w2, a2s, a2d, rep)


def _tc_final(acc2, b2r, batch2d, g):
  n = acc2.shape[1]
  f2 = b2r.shape[1]

  def body(a_ref, b2_ref, bt_ref, o_ref):
    a = a_ref[...]
    s = a[0] + a[1]
    num = s[:, :f2]
    den = s[:, f2:f2 + 1]
    h2o = num * (1.0 / (den + 1e-16)) + b2_ref[...]
    bt = bt_ref[...]
    gi = lax.broadcasted_iota(jnp.int32, (g, n), 0)
    m = (gi == bt).astype(_F32)
    summ = jnp.dot(m, h2o, preferred_element_type=_F32, precision=_HIGH)
    cnt = jnp.sum(m, axis=1, keepdims=True)
    pooled = summ / jnp.maximum(cnt, 1.0)
    mx = jnp.max(pooled, axis=1, keepdims=True)
    ex = jnp.exp(pooled - mx)
    lse = jnp.log(jnp.sum(ex, axis=1, keepdims=True))
    o_ref[...] = pooled - mx - lse

  return pl.pallas_call(
      body,
      out_shape=jax.ShapeDtypeStruct((g, f2), _F32),
  )(acc2, b2r, batch2d)


def _block_diag(att):
  h_, c_ = att.shape[1], att.shape[2]
  eye = jnp.eye(h_, dtype=_F32)
  m = (att[0][:, :, None] * eye[:, None, :]).reshape(h_ * c_, h_)
  return jnp.pad(m, ((0, 0), (0, 16 - h_)))


def kernel(x, edge_index, batch, W1, att_src1, att_dst1, b1,
           W2, att_src2, att_dst2, b2):
  n = x.shape[0]
  f1 = W1.shape[1]
  f2 = W2.shape[1]
  g = 64
  h1n = att_src1.shape[1]

  e = edge_index.shape[1]
  src3a = edge_index[0].reshape(_NW, (e // _NW) // 40, 40)
  dst3a = edge_index[1].reshape(_NW, (e // _NW) // 40, 40)
  src3b = edge_index[0].reshape(_NW, (e // _NW) // _B, _B)
  dst3b = edge_index[1].reshape(_NW, (e // _NW) // _B, _B)
  a1s = _block_diag(att_src1)
  a1d = _block_diag(att_dst1)
  a2s = jnp.tile(att_src2[0, 0][:, None], (1, 16)).astype(_F32)
  a2d = jnp.tile(att_dst2[0, 0][:, None], (1, 16)).astype(_F32)
  rep1 = jnp.repeat(jnp.eye(h1n, dtype=_F32), f1 // h1n, axis=1)
  z1 = jnp.zeros((n, f1 + 16), _F32)
  z2 = jnp.zeros((n, f2 + 16), _F32)
  b1r = b1.reshape(1, f1)
  b2r = b2.reshape(1, f2)
  batch2d = batch.reshape(1, n)

  h1, ss1, sd1 = _tc_prep1(x, W1, a1s, a1d)
  acc1 = _sc_edges(h1, ss1, sd1, src3a, dst3a, z1, heads=h1n)
  h2, ss2, sd2 = _tc_mid(acc1, b1r, W2, a2s, a2d, rep1)
  acc2 = _sc_edges(h2, ss2, sd2, src3b, dst3b, z2, heads=1)
  return _tc_final(acc2, b2r, batch2d, g)


# speedup vs baseline: 82.3320x; 1.4923x over previous
import dataclasses
import functools

import jax
import jax.numpy as jnp
from jax import lax
from jax.experimental import pallas as pl
from jax.experimental.pallas import tpu as pltpu
from jax.experimental.pallas import tpu_sc as plsc

_NC = 2
_NS = 16
_NW = _NC * _NS
_B = 80
_BIG_NEG = -1e30
_F32 = jnp.float32
_HIGH = lax.Precision.HIGHEST


def _tc_prep1(x, w1, a1s, a1d):
  n = x.shape[0]
  f = w1.shape[1]

  def body(x_ref, w_ref, as_ref, ad_ref, h_ref, sd_ref):
    h = jnp.dot(x_ref[...], w_ref[...], preferred_element_type=_F32,
                precision=_HIGH)
    ss = jnp.dot(h, as_ref[...], preferred_element_type=_F32, precision=_HIGH)
    col = lax.broadcasted_iota(jnp.int32, ss.shape, 1)
    ss = jnp.where(col >= 8, _BIG_NEG, ss)
    h_ref[...] = jnp.concatenate([h, ss], axis=1)
    sd_ref[...] = jnp.dot(h, ad_ref[...], preferred_element_type=_F32,
                          precision=_HIGH)

  return pl.pallas_call(
      body,
      out_shape=(
          jax.ShapeDtypeStruct((n, f + 16), _F32),
          jax.ShapeDtypeStruct((n, 16), _F32),
      ),
  )(x, w1, a1s, a1d)


def _sc_edges(haug, sd, src3, dst3, zeros, heads):
  n, ft = haug.shape
  f = ft - 16
  nw, nb, bsz = src3.shape
  cph = f // heads
  mesh = plsc.VectorSubcoreMesh(core_axis_name="c", subcore_axis_name="s")
  cp = pltpu.CompilerParams()
  fields = pltpu.CompilerParams.__dataclass_fields__
  if "needs_layout_passes" in fields:
    cp = dataclasses.replace(cp, needs_layout_passes=False)
  if "use_tc_tiling_on_sc" in fields:
    cp = dataclasses.replace(cp, use_tc_tiling_on_sc=False)

  @functools.partial(
      pl.kernel,
      out_type=jax.ShapeDtypeStruct((_NC, n, ft), _F32),
      mesh=mesh,
      compiler_params=cp,
      scratch_types=[
          pltpu.VMEM_SHARED((n, ft), _F32),
          pltpu.VMEM((8, bsz), jnp.int32),
          pltpu.VMEM((8, bsz), jnp.int32),
          pltpu.VMEM((3, bsz, 16), _F32),
          pltpu.VMEM((3, bsz, ft), _F32),
          pltpu.SemaphoreType.DMA,
          pltpu.SemaphoreType.DMA,
          pltpu.SemaphoreType.DMA,
          pltpu.SemaphoreType.DMA,
          pltpu.SemaphoreType.DMA,
          pltpu.SemaphoreType.DMA,
      ],
  )
  def k(h_hbm, sd_hbm, src_hbm, dst_hbm, z_hbm, out_hbm,
        acc, isrc, idst, drows, orows,
        gs0, gs1, gs2, ssem, ix0, ix1):
    cid = lax.axis_index("c")
    sid = lax.axis_index("s")
    wid = cid * _NS + sid
    gsems = (gs0, gs1, gs2)
    isems = (ix0, ix1)

    @pl.when(sid == 0)
    def _zero():
      pltpu.sync_copy(z_hbm, acc)

    plsc.subcore_barrier()

    mask0 = (lax.iota(jnp.int32, 16) < 1).astype(_F32)

    def i_descs(b):
      j = jnp.bitwise_and(b, 7)
      return (
          pltpu.make_async_copy(src_hbm.at[wid, b], isrc.at[j], isems[0]),
          pltpu.make_async_copy(dst_hbm.at[wid, b], idst.at[j], isems[0]),
      )

    def issue_i(b):
      for d in i_descs(b):
        d.start()

    def wait_i(b):
      for d in i_descs(b):
        d.wait()

    def g_descs(b, s):
      j = jnp.bitwise_and(b, 7)
      return (
          pltpu.make_async_copy(h_hbm.at[isrc.at[j]], orows.at[s], gsems[s]),
          pltpu.make_async_copy(sd_hbm.at[idst.at[j]], drows.at[s], gsems[s]),
      )

    def issue_g(b, s):
      for d in g_descs(b, s):
        d.start()

    def wait_g(b, s):
      for d in g_descs(b, s):
        d.wait()

    def issue_s(b, s):
      j = jnp.bitwise_and(b, 7)
      pltpu.async_copy(orows.at[s], acc.at[idst.at[j]], ssem, add=True)

    def wait_s(b, s):
      j = jnp.bitwise_and(b, 7)
      pltpu.make_async_copy(orows.at[s], acc.at[idst.at[j]], ssem).wait()

    def compute(s):
      @pl.loop(0, bsz, unroll=4)
      def _edge(e):
        a = orows[s, e, pl.ds(f, 16)] + drows[s, e, :]
        w = jnp.exp(jnp.maximum(a, a * 0.2))
        if heads == 1:
          orows[s, e, pl.ds(f, 16)] = w * mask0
        else:
          orows[s, e, pl.ds(f, 16)] = w
        for hd in range(heads):
          if heads == 1:
            sv = w
          else:
            idxk = jnp.full((16,), hd, jnp.int32)
            sv = jnp.take_along_axis(w, idxk, axis=0,
                                     mode="promise_in_bounds")
          for c in range(cph // 16):
            off = hd * cph + c * 16
            orows[s, e, pl.ds(off, 16)] = orows[s, e, pl.ds(off, 16)] * sv

    issue_i(0)
    wait_i(0)
    issue_g(0, 0)
    issue_i(1)
    wait_i(1)
    issue_g(1, 1)
    issue_i(2)

    nbm = nb - (nb % 3)

    @pl.loop(0, nbm, step=3)
    def _triple(i):
      for p in range(3):
        b = i + p
        wait_g(b, p)

        @pl.when(b >= 1)
        def _():
          wait_s(b - 1, (p + 2) % 3)

        @pl.when(b + 2 < nb)
        def _():
          wait_i(b + 2)
          issue_g(b + 2, (p + 2) % 3)

        @pl.when(b + 3 < nb)
        def _():
          issue_i(b + 3)

        compute(p)
        issue_s(b, p)

    for tb in range(nbm, nb):
      p = tb % 3
      wait_g(tb, p)
      wait_s(tb - 1, (p + 2) % 3)
      compute(p)
      issue_s(tb, p)

    wait_s(nb - 1, (nb - 1) % 3)

    plsc.subcore_barrier()

    @pl.when(sid == 0)
    def _writeout():
      pltpu.sync_copy(acc, out_hbm.at[cid])

  return k(haug, sd, src3, dst3, zeros)


def _tc_mid(acc1, b1r, w2, a2s, a2d, rep):
  n = acc1.shape[1]
  f = b1r.shape[1]
  f2 = w2.shape[1]
  heads = rep.shape[0]

  rb = 2000

  def body(a_ref, b1_ref, w2_ref, as_ref, ad_ref, rep_ref,
           h2_ref, sd_ref):
    a = a_ref[...]
    s = a[0] + a[1]
    num = s[:, :f]
    den = s[:, f:f + heads]
    r = 1.0 / (den + 1e-16)
    rf = jnp.dot(r, rep_ref[...], preferred_element_type=_F32,
                 precision=_HIGH)
    o1 = num * rf + b1_ref[...]
    h2 = jnp.dot(o1, w2_ref[...], preferred_element_type=_F32,
                 precision=_HIGH)
    ss = jnp.dot(h2, as_ref[...], preferred_element_type=_F32,
                 precision=_HIGH)
    h2_ref[...] = jnp.concatenate([h2, ss], axis=1)
    sd_ref[...] = jnp.dot(h2, ad_ref[...], preferred_element_type=_F32,
                          precision=_HIGH)

  ft = acc1.shape[2]
  return pl.pallas_call(
      body,
      grid=(n // rb,),
      in_specs=[
          pl.BlockSpec((2, rb, ft), lambda i: (0, i, 0)),
          pl.BlockSpec((1, f), lambda i: (0, 0)),
          pl.BlockSpec(w2.shape, lambda i: (0, 0)),
          pl.BlockSpec(a2s.shape, lambda i: (0, 0)),
          pl.BlockSpec(a2d.shape, lambda i: (0, 0)),
          pl.BlockSpec(rep.shape, lambda i: (0, 0)),
      ],
      out_specs=(
          pl.BlockSpec((rb, f2 + 16), lambda i: (i, 0)),
          pl.BlockSpec((rb, 16), lambda i: (i, 0)),
      ),
      out_shape=(
          jax.ShapeDtypeStruct((n, f2 + 16), _F32),
          jax.ShapeDtypeStruct((n, 16), _F32),
      ),
  )(acc1, b1r, w2, a2s, a2d, rep)


def _tc_final(acc2, b2r, batch2d, g):
  n = acc2.shape[1]
  f2 = b2r.shape[1]

  def body(a_ref, b2_ref, bt_ref, o_ref):
    a = a_ref[...]
    s = a[0] + a[1]
    num = s[:, :f2]
    den = s[:, f2:f2 + 1]
    h2o = num * (1.0 / (den + 1e-16)) + b2_ref[...]
    bt = bt_ref[...]
    gi = lax.broadcasted_iota(jnp.int32, (g, n), 0)
    m = (gi == bt).astype(_F32)
    summ = jnp.dot(m, h2o, preferred_element_type=_F32, precision=_HIGH)
    cnt = jnp.sum(m, axis=1, keepdims=True)
    pooled = summ / jnp.maximum(cnt, 1.0)
    mx = jnp.max(pooled, axis=1, keepdims=True)
    ex = jnp.exp(pooled - mx)
    lse = jnp.log(jnp.sum(ex, axis=1, keepdims=True))
    o_ref[...] = pooled - mx - lse

  return pl.pallas_call(
      body,
      out_shape=jax.ShapeDtypeStruct((g, f2), _F32),
  )(acc2, b2r, batch2d)


def _block_diag(att):
  h_, c_ = att.shape[1], att.shape[2]
  eye = jnp.eye(h_, dtype=_F32)
  m = (att[0][:, :, None] * eye[:, None, :]).reshape(h_ * c_, h_)
  return jnp.pad(m, ((0, 0), (0, 16 - h_)))


def kernel(x, edge_index, batch, W1, att_src1, att_dst1, b1,
           W2, att_src2, att_dst2, b2):
  n = x.shape[0]
  f1 = W1.shape[1]
  f2 = W2.shape[1]
  g = 64
  h1n = att_src1.shape[1]

  e = edge_index.shape[1]
  src3 = edge_index[0].reshape(_NW, (e // _NW) // _B, _B)
  dst3 = edge_index[1].reshape(_NW, (e // _NW) // _B, _B)
  a1s = _block_diag(att_src1)
  a1d = _block_diag(att_dst1)
  a2s = jnp.tile(att_src2[0, 0][:, None], (1, 16)).astype(_F32)
  a2d = jnp.tile(att_dst2[0, 0][:, None], (1, 16)).astype(_F32)
  rep1 = jnp.repeat(jnp.eye(h1n, dtype=_F32), f1 // h1n, axis=1)
  z1 = jnp.zeros((n, f1 + 16), _F32)
  z2 = jnp.zeros((n, f2 + 16), _F32)
  b1r = b1.reshape(1, f1)
  b2r = b2.reshape(1, f2)
  batch2d = batch.reshape(1, n)

  h1aug, sd1 = _tc_prep1(x, W1, a1s, a1d)
  acc1 = _sc_edges(h1aug, sd1, src3, dst3, z1, heads=h1n)
  h2aug, sd2 = _tc_mid(acc1, b1r, W2, a2s, a2d, rep1)
  acc2 = _sc_edges(h2aug, sd2, src3, dst3, z2, heads=1)
  return _tc_final(acc2, b2r, batch2d, g)


# speedup vs baseline: 83.0194x; 1.0083x over previous
import dataclasses
import functools

import jax
import jax.numpy as jnp
from jax import lax
from jax.experimental import pallas as pl
from jax.experimental.pallas import tpu as pltpu
from jax.experimental.pallas import tpu_sc as plsc

_NC = 2
_NS = 16
_NW = _NC * _NS
_B = 80
_BIG_NEG = -1e30
_F32 = jnp.float32
_HIGH = lax.Precision.HIGHEST


def _tc_prep1(x, w1, a1s, a1d):
  n = x.shape[0]
  f = w1.shape[1]

  def body(x_ref, w_ref, as_ref, ad_ref, h_ref, sd_ref):
    h = jnp.dot(x_ref[...], w_ref[...], preferred_element_type=_F32,
                precision=_HIGH)
    ss = jnp.dot(h, as_ref[...], preferred_element_type=_F32, precision=_HIGH)
    col = lax.broadcasted_iota(jnp.int32, ss.shape, 1)
    ss = jnp.where(col >= 8, _BIG_NEG, ss)
    h_ref[...] = jnp.concatenate([h, ss], axis=1)
    sd_ref[...] = jnp.dot(h, ad_ref[...], preferred_element_type=_F32,
                          precision=_HIGH)

  return pl.pallas_call(
      body,
      out_shape=(
          jax.ShapeDtypeStruct((n, f + 16), _F32),
          jax.ShapeDtypeStruct((n, 16), _F32),
      ),
  )(x, w1, a1s, a1d)


def _sc_edges(haug, sd, src3, dst3, zeros, heads):
  n, ft = haug.shape
  f = ft - 16
  nw, nb, bsz = src3.shape
  cph = f // heads
  mesh = plsc.VectorSubcoreMesh(core_axis_name="c", subcore_axis_name="s")
  cp = pltpu.CompilerParams()
  fields = pltpu.CompilerParams.__dataclass_fields__
  if "needs_layout_passes" in fields:
    cp = dataclasses.replace(cp, needs_layout_passes=False)
  if "use_tc_tiling_on_sc" in fields:
    cp = dataclasses.replace(cp, use_tc_tiling_on_sc=False)

  @functools.partial(
      pl.kernel,
      out_type=jax.ShapeDtypeStruct((_NC, n, ft), _F32),
      mesh=mesh,
      compiler_params=cp,
      scratch_types=[
          pltpu.VMEM_SHARED((n, ft), _F32),
          pltpu.VMEM((8, bsz), jnp.int32),
          pltpu.VMEM((8, bsz), jnp.int32),
          pltpu.VMEM((3, bsz, 16), _F32),
          pltpu.VMEM((3, bsz, ft), _F32),
          pltpu.SemaphoreType.DMA,
          pltpu.SemaphoreType.DMA,
          pltpu.SemaphoreType.DMA,
          pltpu.SemaphoreType.DMA,
          pltpu.SemaphoreType.DMA,
          pltpu.SemaphoreType.DMA,
      ],
  )
  def k(h_hbm, sd_hbm, src_hbm, dst_hbm, z_hbm, out_hbm,
        acc, isrc, idst, drows, orows,
        gs0, gs1, gs2, ssem, ix0, ix1):
    cid = lax.axis_index("c")
    sid = lax.axis_index("s")
    wid = cid * _NS + sid
    gsems = (gs0, gs1, gs2)
    isems = (ix0, ix1)

    @pl.when(sid == 0)
    def _zero():
      pltpu.sync_copy(z_hbm, acc)

    plsc.subcore_barrier()

    mask0 = (lax.iota(jnp.int32, 16) < 1).astype(_F32)

    def i_descs(b):
      j = jnp.bitwise_and(b, 7)
      return (
          pltpu.make_async_copy(src_hbm.at[wid, b], isrc.at[j], isems[0]),
          pltpu.make_async_copy(dst_hbm.at[wid, b], idst.at[j], isems[0]),
      )

    def issue_i(b):
      for d in i_descs(b):
        d.start()

    def wait_i(b):
      for d in i_descs(b):
        d.wait()

    def g_descs(b, s):
      j = jnp.bitwise_and(b, 7)
      return (
          pltpu.make_async_copy(h_hbm.at[isrc.at[j]], orows.at[s], gsems[s]),
          pltpu.make_async_copy(sd_hbm.at[idst.at[j]], drows.at[s], gsems[s]),
      )

    def issue_g(b, s):
      for d in g_descs(b, s):
        d.start()

    def wait_g(b, s):
      for d in g_descs(b, s):
        d.wait()

    def issue_s(b, s):
      j = jnp.bitwise_and(b, 7)
      pltpu.async_copy(orows.at[s], acc.at[idst.at[j]], ssem, add=True)

    def wait_s(b, s):
      j = jnp.bitwise_and(b, 7)
      pltpu.make_async_copy(orows.at[s], acc.at[idst.at[j]], ssem).wait()

    def compute(s):
      @pl.loop(0, bsz, unroll=4)
      def _edge(e):
        a = orows[s, e, pl.ds(f, 16)] + drows[s, e, :]
        w = jnp.exp(jnp.maximum(a, a * 0.2))
        if heads == 1:
          orows[s, e, pl.ds(f, 16)] = w * mask0
        else:
          orows[s, e, pl.ds(f, 16)] = w
        for hd in range(heads):
          if heads == 1:
            sv = w
          else:
            idxk = jnp.full((16,), hd, jnp.int32)
            sv = jnp.take_along_axis(w, idxk, axis=0,
                                     mode="promise_in_bounds")
          for c in range(cph // 16):
            off = hd * cph + c * 16
            orows[s, e, pl.ds(off, 16)] = orows[s, e, pl.ds(off, 16)] * sv

    issue_i(0)
    wait_i(0)
    issue_g(0, 0)
    issue_i(1)
    wait_i(1)
    issue_g(1, 1)
    issue_i(2)

    nbm = nb - (nb % 3)

    @pl.loop(0, nbm, step=3)
    def _triple(i):
      for p in range(3):
        b = i + p
        wait_g(b, p)

        @pl.when(b >= 1)
        def _():
          wait_s(b - 1, (p + 2) % 3)

        @pl.when(b + 2 < nb)
        def _():
          wait_i(b + 2)
          issue_g(b + 2, (p + 2) % 3)

        @pl.when(b + 3 < nb)
        def _():
          issue_i(b + 3)

        compute(p)
        issue_s(b, p)

    for tb in range(nbm, nb):
      p = tb % 3
      wait_g(tb, p)
      wait_s(tb - 1, (p + 2) % 3)
      compute(p)
      issue_s(tb, p)

    wait_s(nb - 1, (nb - 1) % 3)

    plsc.subcore_barrier()

    @pl.when(sid == 0)
    def _writeout():
      pltpu.sync_copy(acc, out_hbm.at[cid])

  return k(haug, sd, src3, dst3, zeros)


def _tc_mid(acc1, b1r, w2, a2s, a2d, rep):
  n = acc1.shape[1]
  f = b1r.shape[1]
  f2 = w2.shape[1]
  heads = rep.shape[0]

  rb = 2000

  def body(a_ref, b1_ref, w2_ref, as_ref, ad_ref, rep_ref,
           h2_ref, sd_ref):
    a = a_ref[...]
    s = a[0] + a[1]
    num = s[:, :f]
    den = s[:, f:f + heads]
    r = 1.0 / (den + 1e-16)
    rf = jnp.dot(r, rep_ref[...], preferred_element_type=_F32,
                 precision=_HIGH)
    o1 = num * rf + b1_ref[...]
    h2 = jnp.dot(o1, w2_ref[...], preferred_element_type=_F32,
                 precision=_HIGH)
    ss = jnp.dot(h2, as_ref[...], preferred_element_type=_F32,
                 precision=_HIGH)
    h2_ref[...] = jnp.concatenate([h2, ss], axis=1)
    sd_ref[...] = jnp.dot(h2, ad_ref[...], preferred_element_type=_F32,
                          precision=_HIGH)

  ft = acc1.shape[2]
  return pl.pallas_call(
      body,
      grid=(n // rb,),
      in_specs=[
          pl.BlockSpec((2, rb, ft), lambda i: (0, i, 0)),
          pl.BlockSpec((1, f), lambda i: (0, 0)),
          pl.BlockSpec(w2.shape, lambda i: (0, 0)),
          pl.BlockSpec(a2s.shape, lambda i: (0, 0)),
          pl.BlockSpec(a2d.shape, lambda i: (0, 0)),
          pl.BlockSpec(rep.shape, lambda i: (0, 0)),
      ],
      out_specs=(
          pl.BlockSpec((rb, f2 + 16), lambda i: (i, 0)),
          pl.BlockSpec((rb, 16), lambda i: (i, 0)),
      ),
      out_shape=(
          jax.ShapeDtypeStruct((n, f2 + 16), _F32),
          jax.ShapeDtypeStruct((n, 16), _F32),
      ),
  )(acc1, b1r, w2, a2s, a2d, rep)


def _tc_final(acc2, b2r, batch2d, g):
  n = acc2.shape[1]
  f2 = b2r.shape[1]

  def body(a_ref, b2_ref, bt_ref, o_ref):
    a = a_ref[...]
    s = a[0] + a[1]
    num = s[:, :f2]
    den = s[:, f2:f2 + 1]
    h2o = num * (1.0 / (den + 1e-16)) + b2_ref[...]
    bt = bt_ref[...]
    gi = lax.broadcasted_iota(jnp.int32, (g, n), 0)
    m = (gi == bt).astype(_F32)
    summ = jnp.dot(m, h2o, preferred_element_type=_F32, precision=_HIGH)
    cnt = jnp.sum(m, axis=1, keepdims=True)
    pooled = summ / jnp.maximum(cnt, 1.0)
    mx = jnp.max(pooled, axis=1, keepdims=True)
    ex = jnp.exp(pooled - mx)
    lse = jnp.log(jnp.sum(ex, axis=1, keepdims=True))
    o_ref[...] = pooled - mx - lse

  return pl.pallas_call(
      body,
      out_shape=jax.ShapeDtypeStruct((g, f2), _F32),
  )(acc2, b2r, batch2d)


def _block_diag(att):
  h_, c_ = att.shape[1], att.shape[2]
  eye = jnp.eye(h_, dtype=_F32)
  m = (att[0][:, :, None] * eye[:, None, :]).reshape(h_ * c_, h_)
  return jnp.pad(m, ((0, 0), (0, 16 - h_)))


def kernel(x, edge_index, batch, W1, att_src1, att_dst1, b1,
           W2, att_src2, att_dst2, b2):
  n = x.shape[0]
  f1 = W1.shape[1]
  f2 = W2.shape[1]
  g = 64
  h1n = att_src1.shape[1]

  e = edge_index.shape[1]
  src3 = edge_index[0].reshape(_NW, (e // _NW) // _B, _B)
  dst3 = edge_index[1].reshape(_NW, (e // _NW) // _B, _B)
  b2sz = 200
  src3b = edge_index[0].reshape(_NW, (e // _NW) // b2sz, b2sz)
  dst3b = edge_index[1].reshape(_NW, (e // _NW) // b2sz, b2sz)
  a1s = _block_diag(att_src1)
  a1d = _block_diag(att_dst1)
  a2s = jnp.tile(att_src2[0, 0][:, None], (1, 16)).astype(_F32)
  a2d = jnp.tile(att_dst2[0, 0][:, None], (1, 16)).astype(_F32)
  rep1 = jnp.repeat(jnp.eye(h1n, dtype=_F32), f1 // h1n, axis=1)
  z1 = jnp.zeros((n, f1 + 16), _F32)
  z2 = jnp.zeros((n, f2 + 16), _F32)
  b1r = b1.reshape(1, f1)
  b2r = b2.reshape(1, f2)
  batch2d = batch.reshape(1, n)

  h1aug, sd1 = _tc_prep1(x, W1, a1s, a1d)
  acc1 = _sc_edges(h1aug, sd1, src3, dst3, z1, heads=h1n)
  h2aug, sd2 = _tc_mid(acc1, b1r, W2, a2s, a2d, rep1)
  acc2 = _sc_edges(h2aug, sd2, src3b, dst3b, z2, heads=1)
  return _tc_final(acc2, b2r, batch2d, g)
